# baseline (speedup 1.0000x reference)
import functools

import jax
import jax.numpy as jnp
from jax import lax
from jax.experimental import pallas as pl
from jax.experimental.pallas import tpu as pltpu
from jax.experimental.pallas import tpu_sc as plsc

D = 1024
NC = 2
NS = 16
NW = NC * NS
C = 56


def _make_gather(total_rows):
    b_per_w = total_rows // NW
    nfull = b_per_w // C
    tail = b_per_w - nfull * C
    nhalf = nfull // 2
    mesh = plsc.VectorSubcoreMesh(core_axis_name="c", subcore_axis_name="s")

    @functools.partial(
        pl.kernel,
        mesh=mesh,
        out_type=jax.ShapeDtypeStruct((total_rows, D), jnp.float32),
        scratch_types=[
            pltpu.VMEM((b_per_w,), jnp.int32),
            pltpu.VMEM((C, D), jnp.float32),
            pltpu.VMEM((C, D), jnp.float32),
            pltpu.SemaphoreType.DMA,
            pltpu.SemaphoreType.DMA,
            pltpu.SemaphoreType.DMA,
            pltpu.SemaphoreType.DMA,
        ],
    )
    def gather_kernel(table_hbm, idx_hbm, out_hbm, idx_v, buf0, buf1,
                      g0, g1, o0, o1):
        bufs = (buf0, buf1)
        gsems = (g0, g1)
        osems = (o0, o1)
        wid = lax.axis_index("s") * NC + lax.axis_index("c")
        base = wid * b_per_w
        pltpu.sync_copy(idx_hbm.at[pl.ds(base, b_per_w)], idx_v)

        def g_copy(s, b, n=C):
            return pltpu.make_async_copy(
                table_hbm.at[idx_v.at[pl.ds(s * C, n)]],
                bufs[b].at[pl.ds(0, n)] if n != C else bufs[b],
                gsems[b])

        def o_copy(s, b, n=C):
            return pltpu.make_async_copy(
                bufs[b].at[pl.ds(0, n)] if n != C else bufs[b],
                out_hbm.at[pl.ds(base + s * C, n)],
                osems[b])

        g_copy(0, 0).start()

        def body(i, carry):
            for j in range(2):
                s = 2 * i + j
                b = j

                @pl.when(s >= 1)
                def _(s=s, b=b):
                    o_copy(s - 1, 1 - b).wait()

                @pl.when(s <= nfull - 2)
                def _(s=s, b=b):
                    g_copy(s + 1, 1 - b).start()

                g_copy(s, b).wait()
                o_copy(s, b).start()
            return carry

        lax.fori_loop(0, nhalf, body, 0)
        o_copy(nfull - 1, (nfull - 1) % 2).wait()
        if tail:
            tb = nfull % 2
            g_copy(nfull, tb, tail).start()
            g_copy(nfull, tb, tail).wait()
            o_copy(nfull, tb, tail).start()
            o_copy(nfull, tb, tail).wait()

    return gather_kernel


def kernel(fpe, length, position_ids):
    bsz, seq = position_ids.shape
    idx = position_ids.reshape(-1).astype(jnp.int32)
    out = _make_gather(bsz * seq)(fpe, idx)
    return out.reshape(bsz, seq, fpe.shape[1])


# speedup vs baseline: 1.0008x; 1.0008x over previous
import functools

import jax
import jax.numpy as jnp
from jax import lax
from jax.experimental import pallas as pl
from jax.experimental.pallas import tpu as pltpu
from jax.experimental.pallas import tpu_sc as plsc

D = 1024
NC = 2
NS = 16
NW = NC * NS
CA = 64
CB = 56
PAIR = CA + CB


def _make_gather(total_rows):
    b_per_w = total_rows // NW
    npairs = (b_per_w - CA) // PAIR
    assert npairs * PAIR + CA == b_per_w
    mesh = plsc.VectorSubcoreMesh(core_axis_name="c", subcore_axis_name="s")

    @functools.partial(
        pl.kernel,
        mesh=mesh,
        out_type=jax.ShapeDtypeStruct((total_rows, D), jnp.float32),
        scratch_types=[
            pltpu.VMEM((b_per_w,), jnp.int32),
            pltpu.VMEM((CA, D), jnp.float32),
            pltpu.VMEM((CB, D), jnp.float32),
            pltpu.SemaphoreType.DMA,
            pltpu.SemaphoreType.DMA,
            pltpu.SemaphoreType.DMA,
            pltpu.SemaphoreType.DMA,
        ],
    )
    def gather_kernel(table_hbm, idx_hbm, out_hbm, idx_v, buf_a, buf_b,
                      ga, gb, oa, ob):
        wid = lax.axis_index("s") * NC + lax.axis_index("c")
        base = wid * b_per_w
        pltpu.sync_copy(idx_hbm.at[pl.ds(base, b_per_w)], idx_v)

        def g_a(off):
            return pltpu.make_async_copy(
                table_hbm.at[idx_v.at[pl.ds(off, CA)]], buf_a, ga)

        def o_a(off):
            return pltpu.make_async_copy(
                buf_a, out_hbm.at[pl.ds(base + off, CA)], oa)

        def g_b(off):
            return pltpu.make_async_copy(
                table_hbm.at[idx_v.at[pl.ds(off, CB)]], buf_b, gb)

        def o_b(off):
            return pltpu.make_async_copy(
                buf_b, out_hbm.at[pl.ds(base + off, CB)], ob)

        g_a(0).start()

        def body(i, carry):
            off = PAIR * i

            @pl.when(i > 0)
            def _():
                o_b(off - CB).wait()

            g_b(off + CA).start()
            g_a(off).wait()
            o_a(off).start()

            o_a(off).wait()
            g_a(off + PAIR).start()
            g_b(off + CA).wait()
            o_b(off + CA).start()
            return carry

        lax.fori_loop(0, npairs, body, 0)
        last = npairs * PAIR
        o_b(last - CB).wait()
        g_a(last).wait()
        o_a(last).start()
        o_a(last).wait()

    return gather_kernel


def kernel(fpe, length, position_ids):
    bsz, seq = position_ids.shape
    idx = position_ids.reshape(-1).astype(jnp.int32)
    out = _make_gather(bsz * seq)(fpe, idx)
    return out.reshape(bsz, seq, fpe.shape[1])


# speedup vs baseline: 1.0031x; 1.0023x over previous
import functools

import jax
import jax.numpy as jnp
from jax import lax
from jax.experimental import pallas as pl
from jax.experimental.pallas import tpu as pltpu
from jax.experimental.pallas import tpu_sc as plsc

D = 1024
NC = 2
NS = 16
NW = NC * NS
C = 32


def _make_gather(total_rows):
    b_per_w = total_rows // NW
    nsteps = b_per_w // C
    nhalf = nsteps // 2
    mesh = plsc.VectorSubcoreMesh(core_axis_name="c", subcore_axis_name="s")

    @functools.partial(
        pl.kernel,
        mesh=mesh,
        out_type=jax.ShapeDtypeStruct((total_rows, D), jnp.float32),
        scratch_types=[
            pltpu.VMEM((b_per_w,), jnp.int32),
            pltpu.VMEM((C, D), jnp.float32),
            pltpu.VMEM((C, D), jnp.float32),
            pltpu.SemaphoreType.DMA,
            pltpu.SemaphoreType.DMA,
            pltpu.SemaphoreType.DMA,
            pltpu.SemaphoreType.DMA,
        ],
    )
    def gather_kernel(table_hbm, idx_hbm, out_hbm, idx_v, buf0, buf1,
                      g0, g1, o0, o1):
        bufs = (buf0, buf1)
        gsems = (g0, g1)
        osems = (o0, o1)
        wid = lax.axis_index("s") * NC + lax.axis_index("c")
        base = wid * b_per_w

        def g_copy(s, b):
            return pltpu.make_async_copy(
                table_hbm.at[idx_v.at[pl.ds(s * C, C)]], bufs[b], gsems[b])

        def o_copy(s, b):
            return pltpu.make_async_copy(
                bufs[b], out_hbm.at[pl.ds(base + s * C, C)], osems[b])

        pltpu.sync_copy(idx_hbm.at[pl.ds(base, C)], idx_v.at[pl.ds(0, C)])
        g_copy(0, 0).start()
        pltpu.sync_copy(idx_hbm.at[pl.ds(base + C, b_per_w - C)],
                        idx_v.at[pl.ds(C, b_per_w - C)])

        def body(i, carry):
            for j in range(2):
                s = 2 * i + j
                b = j

                @pl.when(s >= 1)
                def _(s=s, b=b):
                    o_copy(s - 1, 1 - b).wait()

                @pl.when(s <= nsteps - 2)
                def _(s=s, b=b):
                    g_copy(s + 1, 1 - b).start()

                g_copy(s, b).wait()
                o_copy(s, b).start()
            return carry

        lax.fori_loop(0, nhalf, body, 0)
        o_copy(nsteps - 1, (nsteps - 1) % 2).wait()

    return gather_kernel


def kernel(fpe, length, position_ids):
    bsz, seq = position_ids.shape
    idx = position_ids.reshape(-1).astype(jnp.int32)
    out = _make_gather(bsz * seq)(fpe, idx)
    return out.reshape(bsz, seq, fpe.shape[1])
